# baseline (speedup 1.0000x reference)
import functools

import jax
import jax.numpy as jnp
from jax import lax
from jax.experimental import pallas as pl
from jax.experimental.pallas import tpu as pltpu
from jax.experimental.pallas import tpu_sc as plsc

S = 2048
D = 768
E = 64
F = 1536
TM = 128
T_MAX = E + S // TM
P = T_MAX * TM

NC, NS = 2, 16
NW = NC * NS
ROWS_A = P // NW
CHUNK = 64
ROWS_C = S // NW


def _routing_metadata(top_experts):
    e_t = top_experts[:, 0].astype(jnp.int32)
    order = jnp.argsort(e_t, stable=True).astype(jnp.int32)
    sorted_e = jnp.take(e_t, order)
    counts = jnp.zeros((E,), jnp.int32).at[e_t].add(1)
    ntiles = (counts + TM - 1) // TM
    tcum = jnp.cumsum(ntiles)
    tcum_excl = tcum - ntiles
    tile_expert = jnp.minimum(
        jnp.searchsorted(tcum, jnp.arange(T_MAX, dtype=jnp.int32), side="right"),
        E - 1,
    ).astype(jnp.int32)
    counts_excl = jnp.cumsum(counts) - counts
    i = jnp.arange(S, dtype=jnp.int32)
    pos_sorted = TM * jnp.take(tcum_excl, sorted_e) + i - jnp.take(counts_excl, sorted_e)
    src_row = jnp.zeros((P,), jnp.int32).at[pos_sorted].set(order)
    pos = jnp.zeros((S,), jnp.int32).at[order].set(pos_sorted)
    return src_row, pos, tile_expert


def _sc_gather_rows(x, idx, n_rows, base_row):
    mesh = plsc.VectorSubcoreMesh(core_axis_name="c", subcore_axis_name="s")
    per_w = n_rows // NW

    @functools.partial(
        pl.kernel,
        mesh=mesh,
        out_type=jax.ShapeDtypeStruct((n_rows, D), jnp.float32),
        scratch_types=[
            pltpu.VMEM((per_w,), jnp.int32),
            pltpu.VMEM((per_w, D), jnp.float32),
            pltpu.SemaphoreType.DMA,
        ],
    )
    def ka(x_hbm, idx_hbm, out_hbm, idx_v, rows_v, sem):
        wid = lax.axis_index("s") * NC + lax.axis_index("c")
        base = wid * per_w
        pltpu.sync_copy(idx_hbm.at[pl.ds(base_row + base, per_w)], idx_v)
        pltpu.async_copy(x_hbm.at[idx_v], rows_v, sem).wait()
        pltpu.sync_copy(rows_v, out_hbm.at[pl.ds(base, per_w)])

    return ka(x, idx)


def _sc_gather_in(x, src_row):
    parts = [
        _sc_gather_rows(x, src_row, S, k * S) for k in range(P // S)
    ]
    return jnp.concatenate(parts, axis=0)


def _sc_gather_out(ys, pos):
    mesh = plsc.VectorSubcoreMesh(core_axis_name="c", subcore_axis_name="s")

    @functools.partial(
        pl.kernel,
        mesh=mesh,
        out_type=jax.ShapeDtypeStruct((S, D), jnp.float32),
        scratch_types=[
            pltpu.VMEM((ROWS_C,), jnp.int32),
            pltpu.VMEM((ROWS_C, D), jnp.float32),
            pltpu.SemaphoreType.DMA,
        ],
    )
    def kc(ys_hbm, pos_hbm, out_hbm, idx_v, rows_v, sem):
        wid = lax.axis_index("s") * NC + lax.axis_index("c")
        base = wid * ROWS_C
        pltpu.sync_copy(pos_hbm.at[pl.ds(base, ROWS_C)], idx_v)
        pltpu.async_copy(ys_hbm.at[idx_v], rows_v, sem).wait()
        pltpu.sync_copy(rows_v, out_hbm.at[pl.ds(base, ROWS_C)])

    return kc(ys, pos)


def _tc_body(te_ref, xs_ref, gu_ref, dn_ref, ys_ref):
    xt = xs_ref[...]
    gu = gu_ref[0]
    acts = lax.dot_general(
        xt, gu, (((1,), (1,)), ((), ())), preferred_element_type=jnp.float32
    )
    gate = acts[:, :F]
    up = acts[:, F:]
    act = gate * jax.nn.sigmoid(gate) * up
    dn = dn_ref[0]
    ys_ref[...] = lax.dot_general(
        act, dn, (((1,), (1,)), ((), ())), preferred_element_type=jnp.float32
    )


def _tc_grouped_matmul(xs, gate_up_weights, down_weights, tile_expert,
                       interpret=False):
    grid_spec = pltpu.PrefetchScalarGridSpec(
        num_scalar_prefetch=1,
        grid=(T_MAX,),
        in_specs=[
            pl.BlockSpec((TM, D), lambda j, te: (j, 0)),
            pl.BlockSpec((1, 2 * F, D), lambda j, te: (te[j], 0, 0)),
            pl.BlockSpec((1, D, F), lambda j, te: (te[j], 0, 0)),
        ],
        out_specs=pl.BlockSpec((TM, D), lambda j, te: (j, 0)),
    )
    return pl.pallas_call(
        _tc_body,
        grid_spec=grid_spec,
        out_shape=jax.ShapeDtypeStruct((P, D), jnp.float32),
        interpret=interpret,
    )(tile_expert, xs, gate_up_weights, down_weights)


def _scale_body(r_ref, w_ref, o_ref):
    o_ref[...] = r_ref[...] * w_ref[...]


def _tc_scale(rows, tw, interpret=False):
    return pl.pallas_call(
        _scale_body,
        out_shape=jax.ShapeDtypeStruct((S, D), jnp.float32),
        interpret=interpret,
    )(rows, tw.reshape(S, 1))


def kernel(x, weights, top_weights, top_experts, gate_up_weights, down_weights):
    del weights
    q_len = x.shape[1]
    xf = x.reshape(S, D)
    tw = top_weights[:, 0].astype(jnp.float32)

    src_row, pos, tile_expert = _routing_metadata(top_experts)
    xs = _sc_gather_in(xf, src_row)
    ys = _tc_grouped_matmul(xs, gate_up_weights, down_weights, tile_expert)
    rows = _sc_gather_out(ys, pos)
    out = _tc_scale(rows, tw)
    return out.reshape(-1, q_len, D)


# speedup vs baseline: 1.6232x; 1.6232x over previous
import functools

import jax
import jax.numpy as jnp
from jax import lax
from jax.experimental import pallas as pl
from jax.experimental.pallas import tpu as pltpu
from jax.experimental.pallas import tpu_sc as plsc

S = 2048
D = 768
E = 64
F = 1536
TM = 128
T_MAX = E + S // TM
P = T_MAX * TM

NC, NS = 2, 16
NW = NC * NS
ROWS_A = P // NW
CHUNK = 64
ROWS_C = S // NW


def _routing_metadata(top_experts):
    e_t = top_experts[:, 0].astype(jnp.int32)
    order = jnp.argsort(e_t, stable=True).astype(jnp.int32)
    sorted_e = jnp.take(e_t, order)
    counts = jnp.zeros((E,), jnp.int32).at[e_t].add(1)
    ntiles = (counts + TM - 1) // TM
    tcum = jnp.cumsum(ntiles)
    tcum_excl = tcum - ntiles
    tile_expert = jnp.minimum(
        jnp.searchsorted(tcum, jnp.arange(T_MAX, dtype=jnp.int32), side="right"),
        E - 1,
    ).astype(jnp.int32)
    counts_excl = jnp.cumsum(counts) - counts
    i = jnp.arange(S, dtype=jnp.int32)
    pos_sorted = TM * jnp.take(tcum_excl, sorted_e) + i - jnp.take(counts_excl, sorted_e)
    src_row = (jnp.arange(P, dtype=jnp.int32) % S).at[pos_sorted].set(order)
    pos = jnp.zeros((S,), jnp.int32).at[order].set(pos_sorted)
    return src_row, pos, tile_expert


def _sc_gather_rows(x, idx, n_rows, base_row):
    mesh = plsc.VectorSubcoreMesh(core_axis_name="c", subcore_axis_name="s")
    per_w = n_rows // NW

    @functools.partial(
        pl.kernel,
        mesh=mesh,
        out_type=jax.ShapeDtypeStruct((n_rows, D), jnp.float32),
        scratch_types=[
            pltpu.VMEM((per_w,), jnp.int32),
            pltpu.VMEM((per_w, D), jnp.float32),
            pltpu.SemaphoreType.DMA,
        ],
    )
    def ka(x_hbm, idx_hbm, out_hbm, idx_v, rows_v, sem):
        wid = lax.axis_index("s") * NC + lax.axis_index("c")
        base = wid * per_w
        pltpu.sync_copy(idx_hbm.at[pl.ds(base_row + base, per_w)], idx_v)
        pltpu.async_copy(x_hbm.at[idx_v], rows_v, sem).wait()
        pltpu.sync_copy(rows_v, out_hbm.at[pl.ds(base, per_w)])

    return ka(x, idx)


def _sc_gather_in(x, src_row):
    parts = [
        _sc_gather_rows(x, src_row, S, k * S) for k in range(P // S)
    ]
    return jnp.concatenate(parts, axis=0)


def _sc_gather_out(ys, pos):
    mesh = plsc.VectorSubcoreMesh(core_axis_name="c", subcore_axis_name="s")

    @functools.partial(
        pl.kernel,
        mesh=mesh,
        out_type=jax.ShapeDtypeStruct((S, D), jnp.float32),
        scratch_types=[
            pltpu.VMEM((ROWS_C,), jnp.int32),
            pltpu.VMEM((ROWS_C, D), jnp.float32),
            pltpu.SemaphoreType.DMA,
        ],
    )
    def kc(ys_hbm, pos_hbm, out_hbm, idx_v, rows_v, sem):
        wid = lax.axis_index("s") * NC + lax.axis_index("c")
        base = wid * ROWS_C
        pltpu.sync_copy(pos_hbm.at[pl.ds(base, ROWS_C)], idx_v)
        pltpu.async_copy(ys_hbm.at[idx_v], rows_v, sem).wait()
        pltpu.sync_copy(rows_v, out_hbm.at[pl.ds(base, ROWS_C)])

    return kc(ys, pos)


def _tc_body(te_ref, xs_ref, gu_ref, dn_ref, ys_ref):
    xt = xs_ref[...]
    gu = gu_ref[0]
    acts = lax.dot_general(
        xt, gu, (((1,), (1,)), ((), ())), preferred_element_type=jnp.float32
    )
    gate = acts[:, :F]
    up = acts[:, F:]
    act = gate * jax.nn.sigmoid(gate) * up
    dn = dn_ref[0]
    ys_ref[...] = lax.dot_general(
        act, dn, (((1,), (1,)), ((), ())), preferred_element_type=jnp.float32
    )


def _tc_grouped_matmul(xs, gate_up_weights, down_weights, tile_expert,
                       interpret=False):
    grid_spec = pltpu.PrefetchScalarGridSpec(
        num_scalar_prefetch=1,
        grid=(T_MAX,),
        in_specs=[
            pl.BlockSpec((TM, D), lambda j, te: (j, 0)),
            pl.BlockSpec((1, 2 * F, D), lambda j, te: (te[j], 0, 0)),
            pl.BlockSpec((1, D, F), lambda j, te: (te[j], 0, 0)),
        ],
        out_specs=pl.BlockSpec((TM, D), lambda j, te: (j, 0)),
    )
    return pl.pallas_call(
        _tc_body,
        grid_spec=grid_spec,
        out_shape=jax.ShapeDtypeStruct((P, D), jnp.float32),
        interpret=interpret,
    )(tile_expert, xs, gate_up_weights, down_weights)


def _scale_body(r_ref, w_ref, o_ref):
    o_ref[...] = r_ref[...] * w_ref[...]


def _tc_scale(rows, tw, interpret=False):
    return pl.pallas_call(
        _scale_body,
        out_shape=jax.ShapeDtypeStruct((S, D), jnp.float32),
        interpret=interpret,
    )(rows, tw.reshape(S, 1))


def kernel(x, weights, top_weights, top_experts, gate_up_weights, down_weights):
    del weights
    q_len = x.shape[1]
    xf = x.reshape(S, D)
    tw = top_weights[:, 0].astype(jnp.float32)

    src_row, pos, tile_expert = _routing_metadata(top_experts)
    xs = _sc_gather_in(xf, src_row)
    ys = _tc_grouped_matmul(xs, gate_up_weights, down_weights, tile_expert)
    rows = _sc_gather_out(ys, pos)
    out = _tc_scale(rows, tw)
    return out.reshape(-1, q_len, D)


# speedup vs baseline: 1.7261x; 1.0633x over previous
import functools

import jax
import jax.numpy as jnp
from jax import lax
from jax.experimental import pallas as pl
from jax.experimental.pallas import tpu as pltpu
from jax.experimental.pallas import tpu_sc as plsc

S = 2048
D = 768
E = 64
F = 1536
TM = 128
T_MAX = E + S // TM
P = T_MAX * TM

NC, NS = 2, 16
NW = NC * NS
ROWS_A = P // NW
CHUNK = 80
ROWS_C = S // NW


def _routing_metadata(top_experts):
    e_t = top_experts[:, 0].astype(jnp.int32)
    order = jnp.argsort(e_t, stable=True).astype(jnp.int32)
    sorted_e = jnp.take(e_t, order)
    counts = jnp.zeros((E,), jnp.int32).at[e_t].add(1)
    ntiles = (counts + TM - 1) // TM
    tcum = jnp.cumsum(ntiles)
    tcum_excl = tcum - ntiles
    tile_expert = jnp.minimum(
        jnp.searchsorted(tcum, jnp.arange(T_MAX, dtype=jnp.int32), side="right"),
        E - 1,
    ).astype(jnp.int32)
    counts_excl = jnp.cumsum(counts) - counts
    i = jnp.arange(S, dtype=jnp.int32)
    pos_sorted = TM * jnp.take(tcum_excl, sorted_e) + i - jnp.take(counts_excl, sorted_e)
    src_row = (jnp.arange(P, dtype=jnp.int32) % S).at[pos_sorted].set(order)
    pos = jnp.zeros((S,), jnp.int32).at[order].set(pos_sorted)
    return src_row, pos, tile_expert


def _sc_gather_in(x, src_row):
    mesh = plsc.VectorSubcoreMesh(core_axis_name="c", subcore_axis_name="s")

    @functools.partial(
        pl.kernel,
        mesh=mesh,
        out_type=jax.ShapeDtypeStruct((P, D), jnp.float32),
        scratch_types=[
            pltpu.VMEM((CHUNK,), jnp.int32),
            pltpu.VMEM((CHUNK,), jnp.int32),
            pltpu.VMEM((CHUNK,), jnp.int32),
            pltpu.VMEM((CHUNK,), jnp.int32),
            pltpu.VMEM((CHUNK, D), jnp.float32),
            pltpu.VMEM((CHUNK, D), jnp.float32),
            pltpu.SemaphoreType.DMA,
            pltpu.SemaphoreType.DMA,
            pltpu.SemaphoreType.DMA,
            pltpu.SemaphoreType.DMA,
        ],
    )
    def ka(x_hbm, src_hbm, xs_hbm, i0, i1, i2, i3, r0, r1, gs0, gs1, ss0, ss1):
        wid = lax.axis_index("s") * NC + lax.axis_index("c")
        base = wid * ROWS_A
        for c, iv in enumerate([i0, i1, i2, i3]):
            pltpu.sync_copy(src_hbm.at[pl.ds(base + c * CHUNK, CHUNK)], iv)
        g0 = pltpu.async_copy(x_hbm.at[i0], r0, gs0)
        g1 = pltpu.async_copy(x_hbm.at[i1], r1, gs1)
        g0.wait()
        s0 = pltpu.async_copy(r0, xs_hbm.at[pl.ds(base, CHUNK)], ss0)
        g1.wait()
        s1 = pltpu.async_copy(r1, xs_hbm.at[pl.ds(base + CHUNK, CHUNK)], ss1)
        s0.wait()
        g2 = pltpu.async_copy(x_hbm.at[i2], r0, gs0)
        s1.wait()
        g3 = pltpu.async_copy(x_hbm.at[i3], r1, gs1)
        g2.wait()
        s2 = pltpu.async_copy(r0, xs_hbm.at[pl.ds(base + 2 * CHUNK, CHUNK)], ss0)
        g3.wait()
        s3 = pltpu.async_copy(r1, xs_hbm.at[pl.ds(base + 3 * CHUNK, CHUNK)], ss1)
        s2.wait()
        s3.wait()

    return ka(x, src_row)


def _sc_gather_out(ys, pos):
    mesh = plsc.VectorSubcoreMesh(core_axis_name="c", subcore_axis_name="s")

    @functools.partial(
        pl.kernel,
        mesh=mesh,
        out_type=jax.ShapeDtypeStruct((S, D), jnp.float32),
        scratch_types=[
            pltpu.VMEM((ROWS_C,), jnp.int32),
            pltpu.VMEM((ROWS_C, D), jnp.float32),
            pltpu.SemaphoreType.DMA,
        ],
    )
    def kc(ys_hbm, pos_hbm, out_hbm, idx_v, rows_v, sem):
        wid = lax.axis_index("s") * NC + lax.axis_index("c")
        base = wid * ROWS_C
        pltpu.sync_copy(pos_hbm.at[pl.ds(base, ROWS_C)], idx_v)
        pltpu.async_copy(ys_hbm.at[idx_v], rows_v, sem).wait()
        pltpu.sync_copy(rows_v, out_hbm.at[pl.ds(base, ROWS_C)])

    return kc(ys, pos)


def _tc_body(te_ref, xs_ref, gu_ref, dn_ref, ys_ref):
    xt = xs_ref[...]
    gu = gu_ref[0]
    acts = lax.dot_general(
        xt, gu, (((1,), (1,)), ((), ())), preferred_element_type=jnp.float32
    )
    gate = acts[:, :F]
    up = acts[:, F:]
    act = gate * jax.nn.sigmoid(gate) * up
    dn = dn_ref[0]
    ys_ref[...] = lax.dot_general(
        act, dn, (((1,), (1,)), ((), ())), preferred_element_type=jnp.float32
    )


def _tc_grouped_matmul(xs, gate_up_weights, down_weights, tile_expert,
                       interpret=False):
    grid_spec = pltpu.PrefetchScalarGridSpec(
        num_scalar_prefetch=1,
        grid=(T_MAX,),
        in_specs=[
            pl.BlockSpec((TM, D), lambda j, te: (j, 0)),
            pl.BlockSpec((1, 2 * F, D), lambda j, te: (te[j], 0, 0)),
            pl.BlockSpec((1, D, F), lambda j, te: (te[j], 0, 0)),
        ],
        out_specs=pl.BlockSpec((TM, D), lambda j, te: (j, 0)),
    )
    return pl.pallas_call(
        _tc_body,
        grid_spec=grid_spec,
        out_shape=jax.ShapeDtypeStruct((P, D), jnp.float32),
        interpret=interpret,
    )(tile_expert, xs, gate_up_weights, down_weights)


def _scale_body(r_ref, w_ref, o_ref):
    o_ref[...] = r_ref[...] * w_ref[...]


def _tc_scale(rows, tw, interpret=False):
    return pl.pallas_call(
        _scale_body,
        out_shape=jax.ShapeDtypeStruct((S, D), jnp.float32),
        interpret=interpret,
    )(rows, tw.reshape(S, 1))


def kernel(x, weights, top_weights, top_experts, gate_up_weights, down_weights):
    del weights
    q_len = x.shape[1]
    xf = x.reshape(S, D)
    tw = top_weights[:, 0].astype(jnp.float32)

    src_row, pos, tile_expert = _routing_metadata(top_experts)
    xs = _sc_gather_in(xf, src_row)
    ys = _tc_grouped_matmul(xs, gate_up_weights, down_weights, tile_expert)
    rows = _sc_gather_out(ys, pos)
    out = _tc_scale(rows, tw)
    return out.reshape(-1, q_len, D)


# speedup vs baseline: 1.8148x; 1.0514x over previous
import functools

import jax
import jax.numpy as jnp
from jax import lax
from jax.experimental import pallas as pl
from jax.experimental.pallas import tpu as pltpu
from jax.experimental.pallas import tpu_sc as plsc

S = 2048
D = 768
E = 64
F = 1536
TM = 128
T_MAX = E + S // TM
P = T_MAX * TM

NC, NS = 2, 16
NW = NC * NS
ROWS_A = P // NW
CHUNK = 80
ROWS_C = S // NW


def _routing_metadata(top_experts):
    e_t = top_experts[:, 0].astype(jnp.int32)
    order = jnp.argsort(e_t, stable=True).astype(jnp.int32)
    sorted_e = jnp.take(e_t, order)
    counts = jnp.zeros((E,), jnp.int32).at[e_t].add(1)
    ntiles = (counts + TM - 1) // TM
    tcum = jnp.cumsum(ntiles)
    tcum_excl = tcum - ntiles
    tile_expert = jnp.minimum(
        jnp.searchsorted(tcum, jnp.arange(T_MAX, dtype=jnp.int32), side="right"),
        E - 1,
    ).astype(jnp.int32)
    counts_excl = jnp.cumsum(counts) - counts
    i = jnp.arange(S, dtype=jnp.int32)
    pos_sorted = TM * jnp.take(tcum_excl, sorted_e) + i - jnp.take(counts_excl, sorted_e)
    src_row = (jnp.arange(P, dtype=jnp.int32) % S).at[pos_sorted].set(order)
    pos = jnp.zeros((S,), jnp.int32).at[order].set(pos_sorted)
    tile_valid = (jnp.arange(T_MAX, dtype=jnp.int32) < tcum[-1]).astype(jnp.int32)
    return src_row, pos, tile_expert, tile_valid


def _sc_gather_in(x, src_row):
    mesh = plsc.VectorSubcoreMesh(core_axis_name="c", subcore_axis_name="s")

    @functools.partial(
        pl.kernel,
        mesh=mesh,
        out_type=jax.ShapeDtypeStruct((P, D), jnp.float32),
        scratch_types=[
            pltpu.VMEM((CHUNK,), jnp.int32),
            pltpu.VMEM((CHUNK,), jnp.int32),
            pltpu.VMEM((CHUNK,), jnp.int32),
            pltpu.VMEM((CHUNK,), jnp.int32),
            pltpu.VMEM((CHUNK, D), jnp.float32),
            pltpu.VMEM((CHUNK, D), jnp.float32),
            pltpu.SemaphoreType.DMA,
            pltpu.SemaphoreType.DMA,
            pltpu.SemaphoreType.DMA,
            pltpu.SemaphoreType.DMA,
        ],
    )
    def ka(x_hbm, src_hbm, xs_hbm, i0, i1, i2, i3, r0, r1, gs0, gs1, ss0, ss1):
        wid = lax.axis_index("s") * NC + lax.axis_index("c")
        base = wid * ROWS_A
        for c, iv in enumerate([i0, i1, i2, i3]):
            pltpu.sync_copy(src_hbm.at[pl.ds(base + c * CHUNK, CHUNK)], iv)
        g0 = pltpu.async_copy(x_hbm.at[i0], r0, gs0)
        g1 = pltpu.async_copy(x_hbm.at[i1], r1, gs1)
        g0.wait()
        s0 = pltpu.async_copy(r0, xs_hbm.at[pl.ds(base, CHUNK)], ss0)
        g1.wait()
        s1 = pltpu.async_copy(r1, xs_hbm.at[pl.ds(base + CHUNK, CHUNK)], ss1)
        s0.wait()
        g2 = pltpu.async_copy(x_hbm.at[i2], r0, gs0)
        s1.wait()
        g3 = pltpu.async_copy(x_hbm.at[i3], r1, gs1)
        g2.wait()
        s2 = pltpu.async_copy(r0, xs_hbm.at[pl.ds(base + 2 * CHUNK, CHUNK)], ss0)
        g3.wait()
        s3 = pltpu.async_copy(r1, xs_hbm.at[pl.ds(base + 3 * CHUNK, CHUNK)], ss1)
        s2.wait()
        s3.wait()

    return ka(x, src_row)


def _sc_gather_out(ys, pos):
    mesh = plsc.VectorSubcoreMesh(core_axis_name="c", subcore_axis_name="s")

    @functools.partial(
        pl.kernel,
        mesh=mesh,
        out_type=jax.ShapeDtypeStruct((S, D), jnp.float32),
        scratch_types=[
            pltpu.VMEM((ROWS_C,), jnp.int32),
            pltpu.VMEM((ROWS_C, D), jnp.float32),
            pltpu.SemaphoreType.DMA,
        ],
    )
    def kc(ys_hbm, pos_hbm, out_hbm, idx_v, rows_v, sem):
        wid = lax.axis_index("s") * NC + lax.axis_index("c")
        base = wid * ROWS_C
        pltpu.sync_copy(pos_hbm.at[pl.ds(base, ROWS_C)], idx_v)
        pltpu.async_copy(ys_hbm.at[idx_v], rows_v, sem).wait()
        pltpu.sync_copy(rows_v, out_hbm.at[pl.ds(base, ROWS_C)])

    return kc(ys, pos)


def _tc_body(te_ref, tv_ref, xs_ref, gu_ref, dn_ref, ys_ref):
    j = pl.program_id(0)

    @pl.when(tv_ref[j] != 0)
    def _():
        xt = xs_ref[...]
        gu = gu_ref[0]
        acts = lax.dot_general(
            xt, gu, (((1,), (1,)), ((), ())), preferred_element_type=jnp.float32
        )
        gate = acts[:, :F]
        up = acts[:, F:]
        act = gate * jax.nn.sigmoid(gate) * up
        dn = dn_ref[0]
        ys_ref[...] = lax.dot_general(
            act, dn, (((1,), (1,)), ((), ())), preferred_element_type=jnp.float32
        )


def _tc_grouped_matmul(xs, gate_up_weights, down_weights, tile_expert,
                       tile_valid, interpret=False):
    grid_spec = pltpu.PrefetchScalarGridSpec(
        num_scalar_prefetch=2,
        grid=(T_MAX,),
        in_specs=[
            pl.BlockSpec((TM, D), lambda j, te, tv: (j, 0)),
            pl.BlockSpec((1, 2 * F, D), lambda j, te, tv: (te[j], 0, 0)),
            pl.BlockSpec((1, D, F), lambda j, te, tv: (te[j], 0, 0)),
        ],
        out_specs=pl.BlockSpec((TM, D), lambda j, te, tv: (j, 0)),
    )
    return pl.pallas_call(
        _tc_body,
        grid_spec=grid_spec,
        out_shape=jax.ShapeDtypeStruct((P, D), jnp.float32),
        interpret=interpret,
    )(tile_expert, tile_valid, xs, gate_up_weights, down_weights)


def _scale_body(r_ref, w_ref, o_ref):
    o_ref[...] = r_ref[...] * w_ref[...]


def _tc_scale(rows, tw, interpret=False):
    return pl.pallas_call(
        _scale_body,
        out_shape=jax.ShapeDtypeStruct((S, D), jnp.float32),
        interpret=interpret,
    )(rows, tw.reshape(S, 1))


def kernel(x, weights, top_weights, top_experts, gate_up_weights, down_weights):
    del weights
    q_len = x.shape[1]
    xf = x.reshape(S, D)
    tw = top_weights[:, 0].astype(jnp.float32)

    src_row, pos, tile_expert, tile_valid = _routing_metadata(top_experts)
    xs = _sc_gather_in(xf, src_row)
    ys = _tc_grouped_matmul(
        xs, gate_up_weights, down_weights, tile_expert, tile_valid
    )
    rows = _sc_gather_out(ys, pos)
    out = _tc_scale(rows, tw)
    return out.reshape(-1, q_len, D)


# speedup vs baseline: 1.9016x; 1.0479x over previous
import functools

import jax
import jax.numpy as jnp
from jax import lax
from jax.experimental import pallas as pl
from jax.experimental.pallas import tpu as pltpu
from jax.experimental.pallas import tpu_sc as plsc

S = 2048
D = 768
E = 64
F = 1536
TM = 64
T_MAX = E + S // TM
P = T_MAX * TM

NC, NS = 2, 16
NW = NC * NS
ROWS_A = P // NW
CHUNK = 48
ROWS_C = S // NW


def _routing_metadata(top_experts):
    e_t = top_experts[:, 0].astype(jnp.int32)
    order = jnp.argsort(e_t, stable=True).astype(jnp.int32)
    sorted_e = jnp.take(e_t, order)
    counts = jnp.zeros((E,), jnp.int32).at[e_t].add(1)
    ntiles = (counts + TM - 1) // TM
    tcum = jnp.cumsum(ntiles)
    tcum_excl = tcum - ntiles
    tile_expert = jnp.minimum(
        jnp.searchsorted(tcum, jnp.arange(T_MAX, dtype=jnp.int32), side="right"),
        E - 1,
    ).astype(jnp.int32)
    counts_excl = jnp.cumsum(counts) - counts
    i = jnp.arange(S, dtype=jnp.int32)
    pos_sorted = TM * jnp.take(tcum_excl, sorted_e) + i - jnp.take(counts_excl, sorted_e)
    src_row = (jnp.arange(P, dtype=jnp.int32) % S).at[pos_sorted].set(order)
    pos = jnp.zeros((S,), jnp.int32).at[order].set(pos_sorted)
    tile_valid = (jnp.arange(T_MAX, dtype=jnp.int32) < tcum[-1]).astype(jnp.int32)
    return src_row, pos, tile_expert, tile_valid


def _sc_gather_in(x, src_row):
    mesh = plsc.VectorSubcoreMesh(core_axis_name="c", subcore_axis_name="s")

    @functools.partial(
        pl.kernel,
        mesh=mesh,
        out_type=jax.ShapeDtypeStruct((P, D), jnp.float32),
        scratch_types=[
            pltpu.VMEM((CHUNK,), jnp.int32),
            pltpu.VMEM((CHUNK,), jnp.int32),
            pltpu.VMEM((CHUNK,), jnp.int32),
            pltpu.VMEM((CHUNK,), jnp.int32),
            pltpu.VMEM((CHUNK, D), jnp.float32),
            pltpu.VMEM((CHUNK, D), jnp.float32),
            pltpu.SemaphoreType.DMA,
            pltpu.SemaphoreType.DMA,
            pltpu.SemaphoreType.DMA,
            pltpu.SemaphoreType.DMA,
        ],
    )
    def ka(x_hbm, src_hbm, xs_hbm, i0, i1, i2, i3, r0, r1, gs0, gs1, ss0, ss1):
        wid = lax.axis_index("s") * NC + lax.axis_index("c")
        base = wid * ROWS_A
        for c, iv in enumerate([i0, i1, i2, i3]):
            pltpu.sync_copy(src_hbm.at[pl.ds(base + c * CHUNK, CHUNK)], iv)
        g0 = pltpu.async_copy(x_hbm.at[i0], r0, gs0)
        g1 = pltpu.async_copy(x_hbm.at[i1], r1, gs1)
        g0.wait()
        s0 = pltpu.async_copy(r0, xs_hbm.at[pl.ds(base, CHUNK)], ss0)
        g1.wait()
        s1 = pltpu.async_copy(r1, xs_hbm.at[pl.ds(base + CHUNK, CHUNK)], ss1)
        s0.wait()
        g2 = pltpu.async_copy(x_hbm.at[i2], r0, gs0)
        s1.wait()
        g3 = pltpu.async_copy(x_hbm.at[i3], r1, gs1)
        g2.wait()
        s2 = pltpu.async_copy(r0, xs_hbm.at[pl.ds(base + 2 * CHUNK, CHUNK)], ss0)
        g3.wait()
        s3 = pltpu.async_copy(r1, xs_hbm.at[pl.ds(base + 3 * CHUNK, CHUNK)], ss1)
        s2.wait()
        s3.wait()

    return ka(x, src_row)


def _sc_gather_out(ys, pos):
    mesh = plsc.VectorSubcoreMesh(core_axis_name="c", subcore_axis_name="s")

    @functools.partial(
        pl.kernel,
        mesh=mesh,
        out_type=jax.ShapeDtypeStruct((S, D), jnp.float32),
        scratch_types=[
            pltpu.VMEM((ROWS_C,), jnp.int32),
            pltpu.VMEM((ROWS_C, D), jnp.float32),
            pltpu.SemaphoreType.DMA,
        ],
    )
    def kc(ys_hbm, pos_hbm, out_hbm, idx_v, rows_v, sem):
        wid = lax.axis_index("s") * NC + lax.axis_index("c")
        base = wid * ROWS_C
        pltpu.sync_copy(pos_hbm.at[pl.ds(base, ROWS_C)], idx_v)
        pltpu.async_copy(ys_hbm.at[idx_v], rows_v, sem).wait()
        pltpu.sync_copy(rows_v, out_hbm.at[pl.ds(base, ROWS_C)])

    return kc(ys, pos)


def _tc_body(te_ref, tv_ref, xs_ref, gu_ref, dn_ref, ys_ref):
    j = pl.program_id(0)

    @pl.when(tv_ref[j] != 0)
    def _():
        xt = xs_ref[...]
        gu = gu_ref[0]
        acts = lax.dot_general(
            xt, gu, (((1,), (1,)), ((), ())), preferred_element_type=jnp.float32
        )
        gate = acts[:, :F]
        up = acts[:, F:]
        act = gate * jax.nn.sigmoid(gate) * up
        dn = dn_ref[0]
        ys_ref[...] = lax.dot_general(
            act, dn, (((1,), (1,)), ((), ())), preferred_element_type=jnp.float32
        )


def _tc_grouped_matmul(xs, gate_up_weights, down_weights, tile_expert,
                       tile_valid, interpret=False):
    grid_spec = pltpu.PrefetchScalarGridSpec(
        num_scalar_prefetch=2,
        grid=(T_MAX,),
        in_specs=[
            pl.BlockSpec((TM, D), lambda j, te, tv: (j, 0)),
            pl.BlockSpec((1, 2 * F, D), lambda j, te, tv: (te[j], 0, 0)),
            pl.BlockSpec((1, D, F), lambda j, te, tv: (te[j], 0, 0)),
        ],
        out_specs=pl.BlockSpec((TM, D), lambda j, te, tv: (j, 0)),
    )
    return pl.pallas_call(
        _tc_body,
        grid_spec=grid_spec,
        out_shape=jax.ShapeDtypeStruct((P, D), jnp.float32),
        interpret=interpret,
    )(tile_expert, tile_valid, xs, gate_up_weights, down_weights)


def _scale_body(r_ref, w_ref, o_ref):
    o_ref[...] = r_ref[...] * w_ref[...]


def _tc_scale(rows, tw, interpret=False):
    return pl.pallas_call(
        _scale_body,
        out_shape=jax.ShapeDtypeStruct((S, D), jnp.float32),
        interpret=interpret,
    )(rows, tw.reshape(S, 1))


def kernel(x, weights, top_weights, top_experts, gate_up_weights, down_weights):
    del weights
    q_len = x.shape[1]
    xf = x.reshape(S, D)
    tw = top_weights[:, 0].astype(jnp.float32)

    src_row, pos, tile_expert, tile_valid = _routing_metadata(top_experts)
    xs = _sc_gather_in(xf, src_row)
    ys = _tc_grouped_matmul(
        xs, gate_up_weights, down_weights, tile_expert, tile_valid
    )
    rows = _sc_gather_out(ys, pos)
    out = _tc_scale(rows, tw)
    return out.reshape(-1, q_len, D)


# speedup vs baseline: 2.4491x; 1.2879x over previous
import functools

import jax
import jax.numpy as jnp
from jax import lax
from jax.experimental import pallas as pl
from jax.experimental.pallas import tpu as pltpu
from jax.experimental.pallas import tpu_sc as plsc

S = 2048
D = 768
E = 64
F = 1536
TM = 64
T_MAX = E + S // TM
P = T_MAX * TM

NC, NS = 2, 16
NW = NC * NS
ROWS_A = P // NW
CHUNK = 48
ROWS_C = S // NW


def _routing_metadata(top_experts):
    e_t = top_experts[:, 0].astype(jnp.int32)
    onehot = (e_t[:, None] == jnp.arange(E, dtype=jnp.int32)[None, :])
    onehot = onehot.astype(jnp.int32)
    cum = jnp.cumsum(onehot, axis=0)
    prior = jnp.sum(cum * onehot, axis=1) - 1
    counts = cum[-1]
    ntiles = (counts + TM - 1) // TM
    tcum = jnp.cumsum(ntiles)
    tcum_excl = tcum - ntiles
    j = jnp.arange(T_MAX, dtype=jnp.int32)
    tile_expert = jnp.minimum(
        jnp.sum((tcum[None, :] <= j[:, None]).astype(jnp.int32), axis=1),
        E - 1,
    )
    pos = TM * jnp.sum(onehot * tcum_excl[None, :], axis=1) + prior
    tile_valid = (j < tcum[-1]).astype(jnp.int32)
    return pos, tile_expert, tile_valid


def _sc_scatter_in(x, pos):
    mesh = plsc.VectorSubcoreMesh(core_axis_name="c", subcore_axis_name="s")

    @functools.partial(
        pl.kernel,
        mesh=mesh,
        out_type=jax.ShapeDtypeStruct((P, D), jnp.float32),
        scratch_types=[
            pltpu.VMEM((ROWS_C,), jnp.int32),
            pltpu.VMEM((ROWS_C, D), jnp.float32),
            pltpu.SemaphoreType.DMA,
        ],
    )
    def ka(x_hbm, pos_hbm, xs_hbm, idx_v, rows_v, sem):
        wid = lax.axis_index("s") * NC + lax.axis_index("c")
        base = wid * ROWS_C
        pltpu.sync_copy(pos_hbm.at[pl.ds(base, ROWS_C)], idx_v)
        pltpu.sync_copy(x_hbm.at[pl.ds(base, ROWS_C)], rows_v)
        pltpu.async_copy(rows_v, xs_hbm.at[idx_v], sem).wait()

    return ka(x, pos)


def _sc_gather_out(ys, pos):
    mesh = plsc.VectorSubcoreMesh(core_axis_name="c", subcore_axis_name="s")

    @functools.partial(
        pl.kernel,
        mesh=mesh,
        out_type=jax.ShapeDtypeStruct((S, D), jnp.float32),
        scratch_types=[
            pltpu.VMEM((ROWS_C,), jnp.int32),
            pltpu.VMEM((ROWS_C, D), jnp.float32),
            pltpu.SemaphoreType.DMA,
        ],
    )
    def kc(ys_hbm, pos_hbm, out_hbm, idx_v, rows_v, sem):
        wid = lax.axis_index("s") * NC + lax.axis_index("c")
        base = wid * ROWS_C
        pltpu.sync_copy(pos_hbm.at[pl.ds(base, ROWS_C)], idx_v)
        pltpu.async_copy(ys_hbm.at[idx_v], rows_v, sem).wait()
        pltpu.sync_copy(rows_v, out_hbm.at[pl.ds(base, ROWS_C)])

    return kc(ys, pos)


def _tc_body(te_ref, tv_ref, xs_ref, gu_ref, dn_ref, ys_ref):
    j = pl.program_id(0)

    @pl.when(tv_ref[j] != 0)
    def _():
        xt = xs_ref[...]
        gu = gu_ref[0]
        acts = lax.dot_general(
            xt, gu, (((1,), (1,)), ((), ())), preferred_element_type=jnp.float32
        )
        gate = acts[:, :F]
        up = acts[:, F:]
        act = gate * jax.nn.sigmoid(gate) * up
        dn = dn_ref[0]
        ys_ref[...] = lax.dot_general(
            act, dn, (((1,), (1,)), ((), ())), preferred_element_type=jnp.float32
        )


def _tc_grouped_matmul(xs, gate_up_weights, down_weights, tile_expert,
                       tile_valid, interpret=False):
    grid_spec = pltpu.PrefetchScalarGridSpec(
        num_scalar_prefetch=2,
        grid=(T_MAX,),
        in_specs=[
            pl.BlockSpec((TM, D), lambda j, te, tv: (j, 0)),
            pl.BlockSpec((1, 2 * F, D), lambda j, te, tv: (te[j], 0, 0)),
            pl.BlockSpec((1, D, F), lambda j, te, tv: (te[j], 0, 0)),
        ],
        out_specs=pl.BlockSpec((TM, D), lambda j, te, tv: (j, 0)),
    )
    return pl.pallas_call(
        _tc_body,
        grid_spec=grid_spec,
        out_shape=jax.ShapeDtypeStruct((P, D), jnp.float32),
        interpret=interpret,
    )(tile_expert, tile_valid, xs, gate_up_weights, down_weights)


def _scale_body(r_ref, w_ref, o_ref):
    o_ref[...] = r_ref[...] * w_ref[...]


def _tc_scale(rows, tw, interpret=False):
    return pl.pallas_call(
        _scale_body,
        out_shape=jax.ShapeDtypeStruct((S, D), jnp.float32),
        interpret=interpret,
    )(rows, tw.reshape(S, 1))


def kernel(x, weights, top_weights, top_experts, gate_up_weights, down_weights):
    del weights
    q_len = x.shape[1]
    xf = x.reshape(S, D)
    tw = top_weights[:, 0].astype(jnp.float32)

    pos, tile_expert, tile_valid = _routing_metadata(top_experts)
    xs = _sc_scatter_in(xf, pos)
    ys = _tc_grouped_matmul(
        xs, gate_up_weights, down_weights, tile_expert, tile_valid
    )
    rows = _sc_gather_out(ys, pos)
    out = _tc_scale(rows, tw)
    return out.reshape(-1, q_len, D)


# speedup vs baseline: 2.4550x; 1.0024x over previous
import functools

import jax
import jax.numpy as jnp
from jax import lax
from jax.experimental import pallas as pl
from jax.experimental.pallas import tpu as pltpu
from jax.experimental.pallas import tpu_sc as plsc

S = 2048
D = 768
E = 64
F = 1536
TM = 64
T_MAX = E + S // TM
P = T_MAX * TM

NC, NS = 2, 16
NW = NC * NS
ROWS_A = P // NW
CHUNK = 48
ROWS_C = S // NW


def _routing_metadata(top_experts):
    e_t = top_experts[:, 0].astype(jnp.int32)
    onehot = (e_t[:, None] == jnp.arange(E, dtype=jnp.int32)[None, :])
    onehot = onehot.astype(jnp.int32)
    cum = jnp.cumsum(onehot, axis=0)
    prior = jnp.sum(cum * onehot, axis=1) - 1
    counts = cum[-1]
    ntiles = (counts + TM - 1) // TM
    tcum = jnp.cumsum(ntiles)
    tcum_excl = tcum - ntiles
    j = jnp.arange(T_MAX, dtype=jnp.int32)
    tile_expert = jnp.minimum(
        jnp.sum((tcum[None, :] <= j[:, None]).astype(jnp.int32), axis=1),
        E - 1,
    )
    pos = TM * jnp.sum(onehot * tcum_excl[None, :], axis=1) + prior
    tile_valid = (j < tcum[-1]).astype(jnp.int32)
    return pos, tile_expert, tile_valid


def _sc_scatter_in(x, pos):
    mesh = plsc.VectorSubcoreMesh(core_axis_name="c", subcore_axis_name="s")

    @functools.partial(
        pl.kernel,
        mesh=mesh,
        out_type=jax.ShapeDtypeStruct((P, D), jnp.float32),
        scratch_types=[
            pltpu.VMEM((ROWS_C,), jnp.int32),
            pltpu.VMEM((ROWS_C, D), jnp.float32),
            pltpu.SemaphoreType.DMA,
        ],
    )
    def ka(x_hbm, pos_hbm, xs_hbm, idx_v, rows_v, sem):
        wid = lax.axis_index("s") * NC + lax.axis_index("c")
        base = wid * ROWS_C
        pltpu.sync_copy(pos_hbm.at[pl.ds(base, ROWS_C)], idx_v)
        pltpu.sync_copy(x_hbm.at[pl.ds(base, ROWS_C)], rows_v)
        pltpu.async_copy(rows_v, xs_hbm.at[idx_v], sem).wait()

    return ka(x, pos)


def _sc_gather_out(ys, pos):
    mesh = plsc.VectorSubcoreMesh(core_axis_name="c", subcore_axis_name="s")

    @functools.partial(
        pl.kernel,
        mesh=mesh,
        out_type=jax.ShapeDtypeStruct((S, D), jnp.float32),
        scratch_types=[
            pltpu.VMEM((ROWS_C,), jnp.int32),
            pltpu.VMEM((ROWS_C, D), jnp.float32),
            pltpu.SemaphoreType.DMA,
        ],
    )
    def kc(ys_hbm, pos_hbm, out_hbm, idx_v, rows_v, sem):
        wid = lax.axis_index("s") * NC + lax.axis_index("c")
        base = wid * ROWS_C
        pltpu.sync_copy(pos_hbm.at[pl.ds(base, ROWS_C)], idx_v)
        pltpu.async_copy(ys_hbm.at[idx_v], rows_v, sem).wait()
        pltpu.sync_copy(rows_v, out_hbm.at[pl.ds(base, ROWS_C)])

    return kc(ys, pos)


def _tc_body(te_ref, tv_ref, xs_ref, g_ref, u_ref, d1_ref, d2_ref, ys_ref):
    j = pl.program_id(0)

    @pl.when(tv_ref[j] != 0)
    def _():
        xt = xs_ref[...]
        gate = lax.dot_general(
            xt, g_ref[0], (((1,), (1,)), ((), ())),
            preferred_element_type=jnp.float32,
        )
        up = lax.dot_general(
            xt, u_ref[0], (((1,), (1,)), ((), ())),
            preferred_element_type=jnp.float32,
        )
        act = gate * jax.nn.sigmoid(gate) * up
        y1 = lax.dot_general(
            act[:, : F // 2], d1_ref[0], (((1,), (1,)), ((), ())),
            preferred_element_type=jnp.float32,
        )
        y2 = lax.dot_general(
            act[:, F // 2 :], d2_ref[0], (((1,), (1,)), ((), ())),
            preferred_element_type=jnp.float32,
        )
        ys_ref[...] = y1 + y2


def _tc_grouped_matmul(xs, gate_up_weights, down_weights, tile_expert,
                       tile_valid, interpret=False):
    grid_spec = pltpu.PrefetchScalarGridSpec(
        num_scalar_prefetch=2,
        grid=(T_MAX,),
        in_specs=[
            pl.BlockSpec((TM, D), lambda j, te, tv: (j, 0)),
            pl.BlockSpec((1, F, D), lambda j, te, tv: (te[j], 0, 0)),
            pl.BlockSpec((1, F, D), lambda j, te, tv: (te[j], 1, 0)),
            pl.BlockSpec((1, D, F // 2), lambda j, te, tv: (te[j], 0, 0)),
            pl.BlockSpec((1, D, F // 2), lambda j, te, tv: (te[j], 0, 1)),
        ],
        out_specs=pl.BlockSpec((TM, D), lambda j, te, tv: (j, 0)),
    )
    return pl.pallas_call(
        _tc_body,
        grid_spec=grid_spec,
        out_shape=jax.ShapeDtypeStruct((P, D), jnp.float32),
        interpret=interpret,
    )(tile_expert, tile_valid, xs, gate_up_weights, gate_up_weights,
      down_weights, down_weights)


def _scale_body(r_ref, w_ref, o_ref):
    o_ref[...] = r_ref[...] * w_ref[...]


def _tc_scale(rows, tw, interpret=False):
    return pl.pallas_call(
        _scale_body,
        out_shape=jax.ShapeDtypeStruct((S, D), jnp.float32),
        interpret=interpret,
    )(rows, tw.reshape(S, 1))


def kernel(x, weights, top_weights, top_experts, gate_up_weights, down_weights):
    del weights
    q_len = x.shape[1]
    xf = x.reshape(S, D)
    tw = top_weights[:, 0].astype(jnp.float32)

    pos, tile_expert, tile_valid = _routing_metadata(top_experts)
    xs = _sc_scatter_in(xf, pos)
    ys = _tc_grouped_matmul(
        xs, gate_up_weights, down_weights, tile_expert, tile_valid
    )
    rows = _sc_gather_out(ys, pos)
    out = _tc_scale(rows, tw)
    return out.reshape(-1, q_len, D)


# speedup vs baseline: 2.4645x; 1.0039x over previous
import functools

import jax
import jax.numpy as jnp
from jax import lax
from jax.experimental import pallas as pl
from jax.experimental.pallas import tpu as pltpu
from jax.experimental.pallas import tpu_sc as plsc

S = 2048
D = 768
E = 64
F = 1536
TM = 64
T_MAX = E + S // TM
P = T_MAX * TM

NC, NS = 2, 16
NW = NC * NS
ROWS_A = P // NW
CHUNK = 48
ROWS_C = S // NW


def _routing_metadata(top_experts):
    e_t = top_experts[:, 0].astype(jnp.int32)
    onehot = (e_t[:, None] == jnp.arange(E, dtype=jnp.int32)[None, :])
    onehot = onehot.astype(jnp.int32)
    cum = jnp.cumsum(onehot, axis=0)
    prior = jnp.sum(cum * onehot, axis=1) - 1
    counts = cum[-1]
    ntiles = (counts + TM - 1) // TM
    tcum = jnp.cumsum(ntiles)
    tcum_excl = tcum - ntiles
    j = jnp.arange(T_MAX, dtype=jnp.int32)
    tile_expert = jnp.minimum(
        jnp.sum((tcum[None, :] <= j[:, None]).astype(jnp.int32), axis=1),
        E - 1,
    )
    pos = TM * jnp.sum(onehot * tcum_excl[None, :], axis=1) + prior
    tile_valid = (j < tcum[-1]).astype(jnp.int32)
    return pos, tile_expert, tile_valid


def _sc_scatter_in(x, tww, pos):
    mesh = plsc.VectorSubcoreMesh(core_axis_name="c", subcore_axis_name="s")

    @functools.partial(
        pl.kernel,
        mesh=mesh,
        out_type=[
            jax.ShapeDtypeStruct((P, D), jnp.float32),
            jax.ShapeDtypeStruct((P, 128), jnp.float32),
        ],
        scratch_types=[
            pltpu.VMEM((ROWS_C,), jnp.int32),
            pltpu.VMEM((ROWS_C, D), jnp.float32),
            pltpu.VMEM((ROWS_C, 128), jnp.float32),
            pltpu.SemaphoreType.DMA,
            pltpu.SemaphoreType.DMA,
        ],
    )
    def ka(x_hbm, tw_hbm, pos_hbm, xs_hbm, w2_hbm, idx_v, rows_v, w_v, sem, sem2):
        wid = lax.axis_index("s") * NC + lax.axis_index("c")
        base = wid * ROWS_C
        pltpu.sync_copy(pos_hbm.at[pl.ds(base, ROWS_C)], idx_v)
        pltpu.sync_copy(x_hbm.at[pl.ds(base, ROWS_C)], rows_v)
        pltpu.sync_copy(tw_hbm.at[pl.ds(base, ROWS_C)], w_v)
        c1 = pltpu.async_copy(rows_v, xs_hbm.at[idx_v], sem)
        c2 = pltpu.async_copy(w_v, w2_hbm.at[idx_v], sem2)
        c1.wait()
        c2.wait()

    return ka(x, tww, pos)


def _sc_gather_out(ys, pos):
    mesh = plsc.VectorSubcoreMesh(core_axis_name="c", subcore_axis_name="s")

    @functools.partial(
        pl.kernel,
        mesh=mesh,
        out_type=jax.ShapeDtypeStruct((S, D), jnp.float32),
        scratch_types=[
            pltpu.VMEM((ROWS_C,), jnp.int32),
            pltpu.VMEM((ROWS_C, D), jnp.float32),
            pltpu.SemaphoreType.DMA,
        ],
    )
    def kc(ys_hbm, pos_hbm, out_hbm, idx_v, rows_v, sem):
        wid = lax.axis_index("s") * NC + lax.axis_index("c")
        base = wid * ROWS_C
        pltpu.sync_copy(pos_hbm.at[pl.ds(base, ROWS_C)], idx_v)
        pltpu.async_copy(ys_hbm.at[idx_v], rows_v, sem).wait()
        pltpu.sync_copy(rows_v, out_hbm.at[pl.ds(base, ROWS_C)])

    return kc(ys, pos)


def _tc_body(te_ref, tv_ref, xs_ref, w_ref, g_ref, u_ref, d1_ref, d2_ref,
             ys_ref):
    j = pl.program_id(0)

    @pl.when(tv_ref[j] != 0)
    def _():
        xt = xs_ref[...]
        gate = lax.dot_general(
            xt, g_ref[0], (((1,), (1,)), ((), ())),
            preferred_element_type=jnp.float32,
        )
        up = lax.dot_general(
            xt, u_ref[0], (((1,), (1,)), ((), ())),
            preferred_element_type=jnp.float32,
        )
        act = gate * jax.nn.sigmoid(gate) * (up * w_ref[0, :, 0:1])
        y1 = lax.dot_general(
            act[:, : F // 2], d1_ref[0], (((1,), (1,)), ((), ())),
            preferred_element_type=jnp.float32,
        )
        y2 = lax.dot_general(
            act[:, F // 2 :], d2_ref[0], (((1,), (1,)), ((), ())),
            preferred_element_type=jnp.float32,
        )
        ys_ref[...] = y1 + y2


def _tc_grouped_matmul(xs, w2, gate_up_weights, down_weights, tile_expert,
                       tile_valid, interpret=False):
    grid_spec = pltpu.PrefetchScalarGridSpec(
        num_scalar_prefetch=2,
        grid=(T_MAX,),
        in_specs=[
            pl.BlockSpec((TM, D), lambda j, te, tv: (j, 0)),
            pl.BlockSpec((1, TM, 128), lambda j, te, tv: (j, 0, 0)),
            pl.BlockSpec((1, F, D), lambda j, te, tv: (te[j], 0, 0)),
            pl.BlockSpec((1, F, D), lambda j, te, tv: (te[j], 1, 0)),
            pl.BlockSpec((1, D, F // 2), lambda j, te, tv: (te[j], 0, 0)),
            pl.BlockSpec((1, D, F // 2), lambda j, te, tv: (te[j], 0, 1)),
        ],
        out_specs=pl.BlockSpec((TM, D), lambda j, te, tv: (j, 0)),
    )
    return pl.pallas_call(
        _tc_body,
        grid_spec=grid_spec,
        out_shape=jax.ShapeDtypeStruct((P, D), jnp.float32),
        interpret=interpret,
    )(tile_expert, tile_valid, xs, w2.reshape(T_MAX, TM, 128),
      gate_up_weights, gate_up_weights, down_weights, down_weights)


def kernel(x, weights, top_weights, top_experts, gate_up_weights, down_weights):
    del weights
    q_len = x.shape[1]
    xf = x.reshape(S, D)
    tww = jnp.broadcast_to(
        top_weights[:, 0].astype(jnp.float32)[:, None], (S, 128)
    )

    pos, tile_expert, tile_valid = _routing_metadata(top_experts)
    xs, w2 = _sc_scatter_in(xf, tww, pos)
    ys = _tc_grouped_matmul(
        xs, w2, gate_up_weights, down_weights, tile_expert, tile_valid
    )
    out = _sc_gather_out(ys, pos)
    return out.reshape(-1, q_len, D)
